# Initial kernel scaffold; baseline (speedup 1.0000x reference)
import jax
import jax.numpy as jnp
from jax.experimental import pallas as pl


def kernel(x, W1, att_src1, att_dst1, b1, W2, att_src2, att_dst2, b2, Wv, bv, Wu, bu, Wc1, bc1, Wc2, bc2, edge_index, patient_idx, num_patients):
    raise NotImplementedError("write your pallas kernel here")


# speedup vs baseline: 1.0000x; 1.0000x over previous
import jax
import jax.numpy as jnp
from jax.experimental import pallas as pl
from jax.experimental.pallas import tpu as pltpu

N = 10000
HEADS = 8
HID = 128
NPAT = 100
NCLS = 7


def _segment_softmax(e, idx, num_segments):
    m = jax.ops.segment_max(e, idx, num_segments=num_segments)
    m = jnp.where(jnp.isfinite(m), m, 0.0)
    ex = jnp.exp(e - m[idx])
    s = jax.ops.segment_sum(ex, idx, num_segments=num_segments)
    return ex / (s[idx] + 1e-16)


def _gat_layer(x, edge_index, W, att_src, att_dst, bias, heads, hid):
    n = x.shape[0]
    loop = jnp.arange(n, dtype=edge_index.dtype)
    src = jnp.concatenate([edge_index[0], loop])
    dst = jnp.concatenate([edge_index[1], loop])
    h = (x @ W).reshape(n, heads, hid)
    a_src = (h * att_src[None, :, :]).sum(-1)
    a_dst = (h * att_dst[None, :, :]).sum(-1)
    e = a_src[src] + a_dst[dst]
    e = jax.nn.leaky_relu(e, 0.2)
    alpha = _segment_softmax(e, dst, n)
    msg = h[src] * alpha[:, :, None]
    out = jax.ops.segment_sum(msg, dst, num_segments=n)
    return out.reshape(n, heads * hid) + bias


def _head_kernel(hpool_ref, wc1_ref, bc1_ref, wc2_ref, bc2_ref, out_ref):
    z = jnp.maximum(hpool_ref[...] @ wc1_ref[...] + bc1_ref[...], 0.0)
    out_ref[...] = z @ wc2_ref[...] + bc2_ref[...]


def kernel(x, W1, att_src1, att_dst1, b1, W2, att_src2, att_dst2, b2,
           Wv, bv, Wu, bu, Wc1, bc1, Wc2, bc2,
           edge_index, patient_idx, num_patients):
    h = jax.nn.elu(_gat_layer(x, edge_index, W1, att_src1, att_dst1, b1, HEADS, HID))
    h = jax.nn.elu(_gat_layer(h, edge_index, W2, att_src2, att_dst2, b2, 1, HID))
    a = jnp.tanh(h @ Wv + bv)
    a = (a @ Wu + bu).squeeze(-1)
    pidx = patient_idx
    a = _segment_softmax(a, pidx, NPAT)
    h_weighted = h * a[:, None]
    h_pool = jax.ops.segment_sum(h_weighted, pidx, num_segments=NPAT)
    logits = pl.pallas_call(
        _head_kernel,
        out_shape=jax.ShapeDtypeStruct((NPAT, NCLS), jnp.float32),
    )(h_pool, Wc1, bc1[None, :], Wc2, bc2[None, :])
    return (logits, h_pool, a)


# speedup vs baseline: 3.9742x; 3.9741x over previous
import functools
import jax
import jax.numpy as jnp
from jax import lax
from jax.experimental import pallas as pl
from jax.experimental.pallas import tpu as pltpu
from jax.experimental.pallas import tpu_sc as plsc

N = 10000
E = 320000
HEADS = 8
HID = 128
NPAT = 100
NCLS = 7

NP = 10240
TILE = 640
NT = 16
HALF = NP // 2
SCAN = E // 16
CH = 2000
NCH = SCAN // CH
SB = 96
JUNK = HALF
LISTSZ = CH + 2 * SB
ZROWS = HALF // NT

_HIGH = lax.Precision.HIGHEST


def _dot(a, b):
    return jnp.dot(a, b, precision=_HIGH, preferred_element_type=jnp.float32)


def _feat_body(x_ref, w_ref, att_s_ref, att_d_ref, h_ref, as_ref, ad_ref, mx_ref, *, H):
    i = pl.program_id(0)
    h = _dot(x_ref[...], w_ref[...])
    h_ref[...] = h
    h3 = h.reshape(TILE, H, HID)
    a_s = (h3 * att_s_ref[...][None, :, :]).sum(-1)
    a_d = (h3 * att_d_ref[...][None, :, :]).sum(-1)
    as_ref[...] = a_s
    ad_ref[...] = a_d
    new = jnp.concatenate([a_s.max(0, keepdims=True), a_d.max(0, keepdims=True)], 0)

    @pl.when(i == 0)
    def _():
        mx_ref[...] = new

    @pl.when(i != 0)
    def _():
        mx_ref[...] = jnp.maximum(mx_ref[...], new)


def _feat_call(x, W, att_src, att_dst, H, Din):
    return pl.pallas_call(
        functools.partial(_feat_body, H=H),
        grid=(NT,),
        in_specs=[
            pl.BlockSpec((TILE, Din), lambda i: (i, 0)),
            pl.BlockSpec((Din, H * HID), lambda i: (0, 0)),
            pl.BlockSpec((H, HID), lambda i: (0, 0)),
            pl.BlockSpec((H, HID), lambda i: (0, 0)),
        ],
        out_specs=[
            pl.BlockSpec((TILE, H * HID), lambda i: (i, 0)),
            pl.BlockSpec((TILE, H), lambda i: (i, 0)),
            pl.BlockSpec((TILE, H), lambda i: (i, 0)),
            pl.BlockSpec((2, H), lambda i: (0, 0)),
        ],
        out_shape=[
            jax.ShapeDtypeStruct((NP, H * HID), jnp.float32),
            jax.ShapeDtypeStruct((NP, H), jnp.float32),
            jax.ShapeDtypeStruct((NP, H), jnp.float32),
            jax.ShapeDtypeStruct((2, H), jnp.float32),
        ],
    )(x, W, att_src, att_dst)


def _factors_body(as_ref, ad_ref, mh_ref, pd_ref, qd_ref, ws_ref,
                  asp_ref, asq_ref, ad2_ref):
    mh = mh_ref[...]
    a_s = as_ref[...]
    a_d = ad_ref[...]
    ps = jnp.exp(a_s - mh)
    qs = jnp.exp(0.2 * a_s - mh)
    pdv = jnp.exp(a_d - mh)
    qdv = jnp.exp(0.2 * a_d - mh)
    pd_ref[...] = pdv
    qd_ref[...] = qdv
    ws_ref[...] = jnp.where(a_s + a_d > 0, ps * pdv, qs * qdv)
    asp_ref[...] = a_s - mh
    asq_ref[...] = 0.2 * a_s - mh
    ad2_ref[...] = a_d + mh


def _factors_call(a_s, a_d, mh2d, H):
    return pl.pallas_call(
        _factors_body,
        grid=(NT,),
        in_specs=[
            pl.BlockSpec((TILE, H), lambda t: (t, 0)),
            pl.BlockSpec((TILE, H), lambda t: (t, 0)),
            pl.BlockSpec((1, H), lambda t: (0, 0)),
        ],
        out_specs=[
            pl.BlockSpec((TILE, H), lambda t: (t, 0)),
            pl.BlockSpec((TILE, H), lambda t: (t, 0)),
            pl.BlockSpec((TILE, H), lambda t: (t, 0)),
            pl.BlockSpec((TILE, H), lambda t: (t, 0)),
            pl.BlockSpec((TILE, H), lambda t: (t, 0)),
            pl.BlockSpec((TILE, H), lambda t: (t, 0)),
        ],
        out_shape=[
            jax.ShapeDtypeStruct((NP, H), jnp.float32),
            jax.ShapeDtypeStruct((NP, H), jnp.float32),
            jax.ShapeDtypeStruct((NP, H), jnp.float32),
            jax.ShapeDtypeStruct((NP, H), jnp.float32),
            jax.ShapeDtypeStruct((NP, H), jnp.float32),
            jax.ShapeDtypeStruct((NP, H), jnp.float32),
        ],
    )(a_s, a_d, mh2d)


def _col(arr, k, H):
    sel = lax.broadcasted_iota(jnp.int32, (1, H), 1) == k
    return jnp.where(sel, arr, 0.0).sum(1, keepdims=True)


def _tables_body(h_ref, as_ref, mh_ref, gp_ref, gq_ref, *, H):
    k = pl.program_id(1)
    a_s = _col(as_ref[...], k, H)
    mh = _col(mh_ref[...], k, H)[0, 0]
    ps = jnp.exp(a_s - mh)
    qs = jnp.exp(0.2 * a_s - mh)
    h = h_ref[...]
    gp_ref[...] = h * ps
    gq_ref[...] = h * qs


def _tables_call(h, a_s, mh2d, H):
    return pl.pallas_call(
        functools.partial(_tables_body, H=H),
        grid=(NT, H),
        in_specs=[
            pl.BlockSpec((TILE, HID), lambda t, k: (t, k)),
            pl.BlockSpec((TILE, H), lambda t, k: (t, 0)),
            pl.BlockSpec((1, H), lambda t, k: (0, 0)),
        ],
        out_specs=[
            pl.BlockSpec((TILE, HID), lambda t, k: (k * NT + t, 0)),
            pl.BlockSpec((TILE, HID), lambda t, k: (k * NT + t, 0)),
        ],
        out_shape=[
            jax.ShapeDtypeStruct((H * NP, HID), jnp.float32),
            jax.ShapeDtypeStruct((H * NP, HID), jnp.float32),
        ],
    )(h, a_s, mh2d)


def _sc_body(src_h, dst_h, asp_h, asq_h, ad2_h, gp_h, gq_h,
             np_h, nn_h, d_h,
             src_v, dst_v, asv, aqv, adv, dloc, slist, dlist,
             ibuf, rb0, acc, sem0, *, H):
    cid = lax.axis_index("c")
    sid = lax.axis_index("s")
    zf = jnp.zeros((16,), jnp.float32)
    base_lo = cid * HALF

    def head_body(k, _):
        pltpu.sync_copy(asp_h.at[pl.ds(k * NP, NP)], asv)
        pltpu.sync_copy(asq_h.at[pl.ds(k * NP, NP)], aqv)
        pltpu.sync_copy(ad2_h.at[pl.ds(k * NP, NP)], adv)

        for cls in range(2):
            g_h = gp_h if cls == 0 else gq_h
            out_h = np_h if cls == 0 else nn_h

            def zd(i, _):
                dloc[pl.ds(i * 16, 16)] = zf
                return ()

            lax.fori_loop(0, HALF // 16, zd, ())

            def zr(r, _):
                for cc in range(HID // 16):
                    rb0[r, pl.ds(cc * 16, 16)] = zf
                return ()

            lax.fori_loop(0, SB, zr, ())
            for j in range(ZROWS // SB):
                pltpu.sync_copy(rb0, acc.at[pl.ds(sid * ZROWS + j * SB, SB)])
            rem = ZROWS - (ZROWS // SB) * SB
            if rem:
                pltpu.sync_copy(rb0.at[pl.ds(0, rem)],
                                acc.at[pl.ds(sid * ZROWS + ZROWS - rem, rem)])
            plsc.subcore_barrier()

            def chunk_body(ch, _):
                pltpu.sync_copy(src_h.at[pl.ds(sid * SCAN + ch * CH, CH)],
                                src_v)
                pltpu.sync_copy(dst_h.at[pl.ds(sid * SCAN + ch * CH, CH)],
                                dst_v)

                def sb(i, cnt):
                    sv = src_v[pl.ds(i * 16, 16)]
                    dv = dst_v[pl.ds(i * 16, 16)]
                    a1 = plsc.load_gather(asv, [sv])
                    a2 = plsc.load_gather(adv, [dv])
                    pos = (a1 + a2) > 0
                    cmask = pos if cls == 0 else jnp.logical_not(pos)
                    dvl = dv - base_lo
                    inhalf = (dvl >= 0) & (dvl < HALF)
                    m = cmask & inhalf
                    if cls == 0:
                        ev = jnp.exp(a1)
                    else:
                        ev = jnp.exp(plsc.load_gather(aqv, [sv]))
                    plsc.addupdate_scatter(dloc, [dvl], ev, mask=m)
                    gidx = sv + k * NP
                    plsc.store_compressed(slist.at[pl.ds(cnt, 16)], gidx,
                                          mask=m)
                    plsc.store_compressed(dlist.at[pl.ds(cnt, 16)], dvl,
                                          mask=m)
                    return cnt + jnp.max(plsc.all_reduce_population_count(m))

                cnt = lax.fori_loop(0, CH // 16, sb, jnp.int32(0))

                padg = jnp.zeros((16,), jnp.int32)
                padd = jnp.zeros((16,), jnp.int32) + JUNK
                for j in range(SB // 16):
                    slist[pl.ds(cnt + j * 16, 16)] = padg
                    dlist[pl.ds(cnt + j * 16, 16)] = padd
                nb = (cnt + (SB - 1)) // SB

                def bb(b, _):
                    pltpu.async_copy(g_h.at[slist.at[pl.ds(b * SB, SB)]],
                                     rb0, sem0).wait()
                    for j in range(SB // 16):
                        ibuf[pl.ds(j * 16, 16)] = (
                            dlist[pl.ds(b * SB + j * 16, 16)])
                    pltpu.sync_copy(rb0, acc.at[ibuf], add=True)
                    return ()

                lax.fori_loop(0, nb, bb, ())
                return ()

            lax.fori_loop(0, NCH, chunk_body, ())
            plsc.subcore_barrier()
            pltpu.sync_copy(
                dloc,
                d_h.at[pl.ds(((k * 2 + cls) * 32 + cid * 16 + sid) * HALF,
                             HALF)])
            pltpu.sync_copy(
                acc.at[pl.ds(sid * ZROWS, ZROWS)],
                out_h.at[pl.ds(k * NP + cid * HALF + sid * ZROWS, ZROWS)])
        return ()

    lax.fori_loop(0, H, head_body, ())


def _sc_call(src, dst, aspT, asqT, ad2T, Gp, Gq, H):
    mesh = plsc.VectorSubcoreMesh(core_axis_name="c", subcore_axis_name="s",
                                  num_cores=2, num_subcores=16)
    f = pl.kernel(
        functools.partial(_sc_body, H=H),
        out_type=(
            jax.ShapeDtypeStruct((H * NP, HID), jnp.float32),
            jax.ShapeDtypeStruct((H * NP, HID), jnp.float32),
            jax.ShapeDtypeStruct((H * 2 * 32 * HALF,), jnp.float32),
        ),
        mesh=mesh,
        compiler_params=pltpu.CompilerParams(needs_layout_passes=False),
        scratch_types=[
            pltpu.VMEM((CH,), jnp.int32),
            pltpu.VMEM((CH,), jnp.int32),
            pltpu.VMEM((NP,), jnp.float32),
            pltpu.VMEM((NP,), jnp.float32),
            pltpu.VMEM((NP,), jnp.float32),
            pltpu.VMEM((HALF,), jnp.float32),
            pltpu.VMEM((LISTSZ,), jnp.int32),
            pltpu.VMEM((LISTSZ,), jnp.int32),
            pltpu.VMEM((SB,), jnp.int32),
            pltpu.VMEM((SB, HID), jnp.float32),
            pltpu.VMEM_SHARED((HALF + 8, HID), jnp.float32),
            pltpu.SemaphoreType.DMA,
        ],
    )
    return f(src, dst, aspT, asqT, ad2T, Gp, Gq)


def _combine_body(np_ref, nn_ref, dpt_ref, dnt_ref,
                  pd_ref, qd_ref, ws_ref, h_ref, b_ref, out_ref, *, H):
    t = pl.program_id(0)
    k = pl.program_id(1)
    pdv = _col(pd_ref[...], k, H)
    qdv = _col(qd_ref[...], k, H)
    ws = _col(ws_ref[...], k, H)
    num = pdv * np_ref[...] + qdv * nn_ref[...] + ws * h_ref[...]
    dpsum = dpt_ref[...].sum(2).reshape(TILE, 1)
    dnsum = dnt_ref[...].sum(2).reshape(TILE, 1)
    den = pdv * dpsum + qdv * dnsum + ws
    rsel = lax.broadcasted_iota(jnp.int32, (H, 1), 0) == k
    brow = jnp.where(rsel, b_ref[...], 0.0).sum(0, keepdims=True)
    o = num / den + brow
    o = jnp.where(o > 0, o, jnp.exp(jnp.minimum(o, 0.0)) - 1.0)
    ridx = t * TILE + lax.broadcasted_iota(jnp.int32, (TILE, 1), 0)
    out_ref[...] = jnp.where(ridx < N, o, 0.0)


def _combine_call(Np, Nn, DpT, DnT, pd, qd, ws, h, b2d, H):
    return pl.pallas_call(
        functools.partial(_combine_body, H=H),
        grid=(NT, H),
        in_specs=[
            pl.BlockSpec((TILE, HID), lambda t, k: (k * NT + t, 0)),
            pl.BlockSpec((TILE, HID), lambda t, k: (k * NT + t, 0)),
            pl.BlockSpec((1, TILE, NT), lambda t, k: (k, t, 0)),
            pl.BlockSpec((1, TILE, NT), lambda t, k: (k, t, 0)),
            pl.BlockSpec((TILE, H), lambda t, k: (t, 0)),
            pl.BlockSpec((TILE, H), lambda t, k: (t, 0)),
            pl.BlockSpec((TILE, H), lambda t, k: (t, 0)),
            pl.BlockSpec((TILE, HID), lambda t, k: (t, k)),
            pl.BlockSpec((H, HID), lambda t, k: (0, 0)),
        ],
        out_specs=pl.BlockSpec((TILE, HID), lambda t, k: (t, k)),
        out_shape=jax.ShapeDtypeStruct((NP, H * HID), jnp.float32),
    )(Np, Nn, DpT, DnT, pd, qd, ws, h, b2d)


def _mil_body(h_ref, pidx_ref, wv_ref, bv_ref, wu_ref, bu_ref,
              wc1_ref, bc1_ref, wc2_ref, bc2_ref,
              a_ref, hp_ref, lg_ref):
    h = h_ref[...]
    att = jnp.tanh(_dot(h, wv_ref[...]) + bv_ref[...])
    ar = _dot(att, wu_ref[...]) + bu_ref[...]
    pid = pidx_ref[...]
    iot = lax.broadcasted_iota(jnp.int32, (N, NPAT), 1)
    onehot = iot == pid
    neg = jnp.float32(-jnp.inf)
    mp = jnp.where(onehot, ar, neg).max(0, keepdims=True)
    mp = jnp.where(jnp.isfinite(mp), mp, 0.0)
    mnode = jnp.where(onehot, mp, neg).max(1, keepdims=True)
    ex = jnp.exp(ar - mnode)
    srow = jnp.where(onehot, ex, 0.0).sum(0, keepdims=True)
    snode = jnp.where(onehot, srow, 0.0).sum(1, keepdims=True)
    alpha = ex / (snode + 1e-16)
    a_ref[...] = alpha
    hw = h * alpha
    ohf = onehot.astype(jnp.float32)
    hp = lax.dot_general(ohf, hw, (((0,), (0,)), ((), ())),
                         precision=_HIGH, preferred_element_type=jnp.float32)
    hp_ref[...] = hp
    z = jnp.maximum(_dot(hp, wc1_ref[...]) + bc1_ref[...], 0.0)
    lg_ref[...] = _dot(z, wc2_ref[...]) + bc2_ref[...]


def _mil_call(h2, pidx2, Wv, bv, Wu, bu, Wc1, bc1, Wc2, bc2):
    return pl.pallas_call(
        _mil_body,
        out_shape=[
            jax.ShapeDtypeStruct((N, 1), jnp.float32),
            jax.ShapeDtypeStruct((NPAT, HID), jnp.float32),
            jax.ShapeDtypeStruct((NPAT, NCLS), jnp.float32),
        ],
    )(h2, pidx2, Wv, bv[None, :], Wu, bu[None, :],
      Wc1, bc1[None, :], Wc2, bc2[None, :])


def _gat_layer_opt(x_pad, src, dst, W, att_src, att_dst, bias, H, Din):
    h, a_s, a_d, mx = _feat_call(x_pad, W, att_src, att_dst, H, Din)
    mh = jax.nn.leaky_relu(mx[0] + mx[1], 0.2) * 0.5
    mh2d = mh[None, :]
    pd, qd, ws, asp, asq, ad2 = _factors_call(a_s, a_d, mh2d, H)
    Gp, Gq = _tables_call(h, a_s, mh2d, H)
    aspT = asp.T.reshape(-1)
    asqT = asq.T.reshape(-1)
    ad2T = ad2.T.reshape(-1)
    Npf, Nnf, Df = _sc_call(src, dst, aspT, asqT, ad2T, Gp, Gq, H)
    Dr = Df.reshape(H, 2, 2, NT, HALF).transpose(0, 1, 2, 4, 3)
    Dr = Dr.reshape(H, 2, NP, NT)
    DpT = Dr[:, 0]
    DnT = Dr[:, 1]
    return _combine_call(Npf, Nnf, DpT, DnT, pd, qd, ws, h,
                         bias.reshape(H, HID), H)


def kernel(x, W1, att_src1, att_dst1, b1, W2, att_src2, att_dst2, b2,
           Wv, bv, Wu, bu, Wc1, bc1, Wc2, bc2,
           edge_index, patient_idx, num_patients):
    src = edge_index[0]
    dst = edge_index[1]
    x_pad = jnp.pad(x, ((0, NP - N), (0, 0)))
    h1 = _gat_layer_opt(x_pad, src, dst, W1, att_src1, att_dst1, b1,
                        HEADS, HID)
    h2p = _gat_layer_opt(h1, src, dst, W2, att_src2, att_dst2, b2,
                         1, HEADS * HID)
    h2 = h2p[:N]
    pidx = patient_idx + (jnp.asarray(num_patients, patient_idx.dtype)
                          - jnp.asarray(NPAT, patient_idx.dtype))
    a2, h_pool, logits = _mil_call(h2, pidx[:, None], Wv, bv, Wu, bu,
                                   Wc1, bc1, Wc2, bc2)
    return (logits, h_pool, a2[:, 0])
